# Initial kernel scaffold; baseline (speedup 1.0000x reference)
import jax
import jax.numpy as jnp
from jax.experimental import pallas as pl


def kernel(node_feats, edge_index, edge_feats, W, b, gamma, beta):
    raise NotImplementedError("write your pallas kernel here")


# speedup vs baseline: 2.1445x; 2.1445x over previous
import functools
import jax
import jax.numpy as jnp
from jax import lax
from jax.experimental import pallas as pl
from jax.experimental.pallas import tpu as pltpu
from jax.experimental.pallas import tpu_sc as plsc

N_NODES = 10000
N_EDGES = 320000
H = 128
E_DIM = 16
OUT_DIM = 2 * H

NC = 2
NS = 16
NW = NC * NS
EDGES_PER_TILE = N_EDGES // NW
CHUNK = 80
NCHUNK = EDGES_PER_TILE // CHUNK

BLK_N = 1000


def _node_mm_body(nf_ref, w_ref, p1_ref, p2_ref):
    nf = nf_ref[...]
    p1_ref[...] = jnp.dot(nf, w_ref[:, :OUT_DIM],
                          preferred_element_type=jnp.float32)
    p2_ref[...] = jnp.dot(nf, w_ref[:, OUT_DIM:],
                          preferred_element_type=jnp.float32)


def _node_matmul(nf, w2):
    grid = N_NODES // BLK_N
    return pl.pallas_call(
        _node_mm_body,
        grid=(grid,),
        in_specs=[
            pl.BlockSpec((BLK_N, H), lambda i: (i, 0)),
            pl.BlockSpec((H, 2 * OUT_DIM), lambda i: (0, 0)),
        ],
        out_specs=[
            pl.BlockSpec((BLK_N, OUT_DIM), lambda i: (i, 0)),
            pl.BlockSpec((BLK_N, OUT_DIM), lambda i: (i, 0)),
        ],
        out_shape=[
            jax.ShapeDtypeStruct((N_NODES, OUT_DIM), jnp.float32),
            jax.ShapeDtypeStruct((N_NODES, OUT_DIM), jnp.float32),
        ],
    )(nf, w2)


def _gather_body(p1_hbm, p2_hbm, src_hbm, dst_hbm, zpart_hbm,
                 sidx, didx, zbuf, gbuf, sem):
    cid = lax.axis_index("c")
    sid = lax.axis_index("s")
    wid = sid * NC + cid
    base = wid * EDGES_PER_TILE

    pltpu.sync_copy(src_hbm.at[wid], sidx)
    pltpu.sync_copy(dst_hbm.at[wid], didx)
    def chunk(j, carry):
        pltpu.async_copy(p1_hbm.at[sidx.at[j]], zbuf, sem).wait()
        pltpu.async_copy(p2_hbm.at[didx.at[j]], gbuf, sem).wait()

        def add_row(r, c2):
            for k in range(OUT_DIM // 16):
                sl = pl.ds(k * 16, 16)
                zbuf[r, sl] += gbuf[r, sl]
            return c2

        lax.fori_loop(0, CHUNK, add_row, 0)
        pltpu.sync_copy(zbuf, zpart_hbm.at[pl.ds(base + j * CHUNK, CHUNK)])
        return carry

    lax.fori_loop(0, NCHUNK, chunk, 0)


def _sc_gather(p1, p2, src3d, dst3d):
    mesh = plsc.VectorSubcoreMesh(core_axis_name="c", subcore_axis_name="s")
    f = pl.kernel(
        _gather_body,
        out_type=jax.ShapeDtypeStruct((N_EDGES, OUT_DIM), jnp.float32),
        mesh=mesh,
        scratch_types=[
            pltpu.VMEM((NCHUNK, CHUNK), jnp.int32),
            pltpu.VMEM((NCHUNK, CHUNK), jnp.int32),
            pltpu.VMEM((CHUNK, OUT_DIM), jnp.float32),
            pltpu.VMEM((CHUNK, OUT_DIM), jnp.float32),
            pltpu.SemaphoreType.DMA,
        ],
    )
    return f(p1, p2, src3d, dst3d)


BLK_E = 2000
E_GRID = N_EDGES // BLK_E


def _edge_term(zpart, ef, w3, bvec):
    return zpart + jnp.dot(ef, w3, preferred_element_type=jnp.float32) + bvec


def _stats_body(zpart_ref, ef_ref, w3_ref, b_ref, out_ref, acc):
    i = pl.program_id(0)

    @pl.when(i == 0)
    def _():
        acc[...] = jnp.zeros_like(acc)

    z = _edge_term(zpart_ref[...], ef_ref[...], w3_ref[...], b_ref[...])
    acc[0:1, :] += jnp.sum(z, axis=0, keepdims=True)
    acc[1:2, :] += jnp.sum(z * z, axis=0, keepdims=True)

    @pl.when(i == E_GRID - 1)
    def _():
        out_ref[...] = acc[...]


def _stats(zpart, ef, w3, bvec):
    return pl.pallas_call(
        _stats_body,
        grid=(E_GRID,),
        in_specs=[
            pl.BlockSpec((BLK_E, OUT_DIM), lambda i: (i, 0)),
            pl.BlockSpec((BLK_E, E_DIM), lambda i: (i, 0)),
            pl.BlockSpec((E_DIM, OUT_DIM), lambda i: (0, 0)),
            pl.BlockSpec((1, OUT_DIM), lambda i: (0, 0)),
        ],
        out_specs=pl.BlockSpec((8, OUT_DIM), lambda i: (0, 0)),
        out_shape=jax.ShapeDtypeStruct((8, OUT_DIM), jnp.float32),
        scratch_shapes=[pltpu.VMEM((8, OUT_DIM), jnp.float32)],
    )(zpart, ef, w3, bvec)


def _msg_body(zpart_ref, ef_ref, w3_ref, b_ref, stats_ref, gam_ref, bet_ref,
              msg_ref):
    z = _edge_term(zpart_ref[...], ef_ref[...], w3_ref[...], b_ref[...])
    mean = stats_ref[0:1, :] * (1.0 / N_EDGES)
    var = stats_ref[1:2, :] * (1.0 / N_EDGES) - mean * mean
    scale = gam_ref[...] * lax.rsqrt(var + 1e-5)
    shift = bet_ref[...] - mean * scale
    zn = z * scale + shift
    sig = jax.nn.sigmoid(zn[:, :H])
    xp = zn[:, H:]
    sp = jnp.maximum(xp, 0.0) + jnp.log1p(jnp.exp(-jnp.abs(xp)))
    msg_ref[...] = sig * sp


def _msg(zpart, ef, w3, bvec, stats, gamma, beta):
    return pl.pallas_call(
        _msg_body,
        grid=(E_GRID,),
        in_specs=[
            pl.BlockSpec((BLK_E, OUT_DIM), lambda i: (i, 0)),
            pl.BlockSpec((BLK_E, E_DIM), lambda i: (i, 0)),
            pl.BlockSpec((E_DIM, OUT_DIM), lambda i: (0, 0)),
            pl.BlockSpec((1, OUT_DIM), lambda i: (0, 0)),
            pl.BlockSpec((8, OUT_DIM), lambda i: (0, 0)),
            pl.BlockSpec((1, OUT_DIM), lambda i: (0, 0)),
            pl.BlockSpec((1, OUT_DIM), lambda i: (0, 0)),
        ],
        out_specs=pl.BlockSpec((BLK_E, H), lambda i: (i, 0)),
        out_shape=jax.ShapeDtypeStruct((N_EDGES, H), jnp.float32),
    )(zpart, ef, w3, bvec, stats, gamma, beta)


N_NODES_PAD = 10240
ROWS_PER_TILE = N_NODES_PAD // NS


def _scatter_body(msg_hbm, src_hbm, zeros_hbm, parts_hbm, sidx, mbuf, acc, sem):
    cid = lax.axis_index("c")
    sid = lax.axis_index("s")
    wid = sid * NC + cid
    base = wid * EDGES_PER_TILE
    stripe = sid * ROWS_PER_TILE

    pltpu.sync_copy(src_hbm.at[wid], sidx)
    pltpu.sync_copy(zeros_hbm.at[pl.ds(stripe, ROWS_PER_TILE)],
                    acc.at[pl.ds(stripe, ROWS_PER_TILE)])
    plsc.subcore_barrier()

    def chunk(j, carry):
        pltpu.async_copy(msg_hbm.at[pl.ds(base + j * CHUNK, CHUNK)], mbuf,
                         sem).wait()
        pltpu.sync_copy(mbuf, acc.at[sidx.at[j]], add=True)
        return carry

    lax.fori_loop(0, NCHUNK, chunk, 0)
    plsc.subcore_barrier()
    pltpu.sync_copy(acc.at[pl.ds(stripe, ROWS_PER_TILE)],
                    parts_hbm.at[cid, pl.ds(stripe, ROWS_PER_TILE)])


def _sc_scatter(msg, src3d, zeros):
    mesh = plsc.VectorSubcoreMesh(core_axis_name="c", subcore_axis_name="s")
    f = pl.kernel(
        _scatter_body,
        out_type=jax.ShapeDtypeStruct((NC, N_NODES_PAD, H), jnp.float32),
        mesh=mesh,
        scratch_types=[
            pltpu.VMEM((NCHUNK, CHUNK), jnp.int32),
            pltpu.VMEM((CHUNK, H), jnp.float32),
            pltpu.VMEM_SHARED((N_NODES_PAD, H), jnp.float32),
            pltpu.SemaphoreType.DMA,
        ],
    )
    return f(msg, src3d, zeros)


def _final_body(nf_ref, p0_ref, p1_ref, out_ref):
    out_ref[...] = nf_ref[...] + p0_ref[0] + p1_ref[0]


def _final_add(nf, parts):
    grid = N_NODES // BLK_N
    return pl.pallas_call(
        _final_body,
        grid=(grid,),
        in_specs=[
            pl.BlockSpec((BLK_N, H), lambda i: (i, 0)),
            pl.BlockSpec((1, BLK_N, H), lambda i: (0, i, 0)),
            pl.BlockSpec((1, BLK_N, H), lambda i: (1, i, 0)),
        ],
        out_specs=pl.BlockSpec((BLK_N, H), lambda i: (i, 0)),
        out_shape=jax.ShapeDtypeStruct((N_NODES, H), jnp.float32),
    )(nf, parts, parts)


@jax.jit
def kernel(node_feats, edge_index, edge_feats, W, b, gamma, beta):
    src = edge_index[0].astype(jnp.int32)
    dst = edge_index[1].astype(jnp.int32)
    wt = W.T
    w12 = jnp.concatenate([wt[:H], wt[H:2 * H]], axis=1)
    w3 = wt[2 * H:]
    bvec = b.reshape(1, OUT_DIM)
    gam = gamma.reshape(1, OUT_DIM)
    bet = beta.reshape(1, OUT_DIM)
    src3d = src.reshape(NW, NCHUNK, CHUNK)
    dst3d = dst.reshape(NW, NCHUNK, CHUNK)

    p1, p2 = _node_matmul(node_feats, w12)
    zpart = _sc_gather(p1, p2, src3d, dst3d)
    stats = _stats(zpart, edge_feats, w3, bvec)
    msg = _msg(zpart, edge_feats, w3, bvec, stats, gam, bet)
    zeros = jnp.zeros((N_NODES_PAD, H), jnp.float32)
    parts = _sc_scatter(msg, src3d, zeros)
    return _final_add(node_feats, parts)


# speedup vs baseline: 2.9184x; 1.3609x over previous
import functools
import jax
import jax.numpy as jnp
from jax import lax
from jax.experimental import pallas as pl
from jax.experimental.pallas import tpu as pltpu
from jax.experimental.pallas import tpu_sc as plsc

N_NODES = 10000
N_EDGES = 320000
H = 128
E_DIM = 16
OUT_DIM = 2 * H

NC = 2
NS = 16
NW = NC * NS
EDGES_PER_TILE = N_EDGES // NW
CHUNK = 80
NCHUNK = EDGES_PER_TILE // CHUNK

BLK_N = 1000


def _node_mm_body(nf_ref, w_ref, p1_ref, p2_ref):
    nf = nf_ref[...]
    p1_ref[...] = jnp.dot(nf, w_ref[:, :OUT_DIM],
                          preferred_element_type=jnp.float32)
    p2_ref[...] = jnp.dot(nf, w_ref[:, OUT_DIM:],
                          preferred_element_type=jnp.float32)


def _node_matmul(nf, w2):
    grid = N_NODES // BLK_N
    return pl.pallas_call(
        _node_mm_body,
        grid=(grid,),
        in_specs=[
            pl.BlockSpec((BLK_N, H), lambda i: (i, 0)),
            pl.BlockSpec((H, 2 * OUT_DIM), lambda i: (0, 0)),
        ],
        out_specs=[
            pl.BlockSpec((BLK_N, OUT_DIM), lambda i: (i, 0)),
            pl.BlockSpec((BLK_N, OUT_DIM), lambda i: (i, 0)),
        ],
        out_shape=[
            jax.ShapeDtypeStruct((N_NODES, OUT_DIM), jnp.float32),
            jax.ShapeDtypeStruct((N_NODES, OUT_DIM), jnp.float32),
        ],
    )(nf, w2)


def _gather_body(p1_hbm, p2_hbm, src_hbm, dst_hbm, zpart_hbm,
                 sidx, didx, zbuf, gbuf, sem):
    cid = lax.axis_index("c")
    sid = lax.axis_index("s")
    wid = sid * NC + cid
    base = wid * EDGES_PER_TILE

    pltpu.sync_copy(src_hbm.at[wid], sidx)
    pltpu.sync_copy(dst_hbm.at[wid], didx)

    def fire(j, s):
        pltpu.async_copy(p1_hbm.at[sidx.at[j]], zbuf.at[s], sem.at[s])
        pltpu.async_copy(p2_hbm.at[didx.at[j]], gbuf.at[s], sem.at[s])

    def drain(j, s):
        pltpu.make_async_copy(p1_hbm.at[sidx.at[j]], zbuf.at[s], sem.at[s]).wait()
        pltpu.make_async_copy(p2_hbm.at[didx.at[j]], gbuf.at[s], sem.at[s]).wait()

    def process(j, s):
        drain(j, s)

        @plsc.parallel_loop(0, CHUNK)
        def _(r):
            for k in range(OUT_DIM // 16):
                sl = pl.ds(k * 16, 16)
                zbuf[s, r, sl] += gbuf[s, r, sl]

        pltpu.sync_copy(zbuf.at[s],
                        zpart_hbm.at[pl.ds(base + j * CHUNK, CHUNK)])

    fire(0, 0)
    fire(1, 1)

    def step(j2, carry):
        for b in range(2):
            j = 2 * j2 + b
            process(j, b)

            @pl.when(j < NCHUNK - 2)
            def _():
                fire(j + 2, b)
        return carry

    lax.fori_loop(0, (NCHUNK - 1) // 2, step, 0)
    process(NCHUNK - 1, (NCHUNK - 1) % 2)


def _sc_gather(p1, p2, src3d, dst3d):
    mesh = plsc.VectorSubcoreMesh(core_axis_name="c", subcore_axis_name="s")
    f = pl.kernel(
        _gather_body,
        out_type=jax.ShapeDtypeStruct((N_EDGES, OUT_DIM), jnp.float32),
        mesh=mesh,
        scratch_types=[
            pltpu.VMEM((NCHUNK, CHUNK), jnp.int32),
            pltpu.VMEM((NCHUNK, CHUNK), jnp.int32),
            pltpu.VMEM((2, CHUNK, OUT_DIM), jnp.float32),
            pltpu.VMEM((2, CHUNK, OUT_DIM), jnp.float32),
            pltpu.SemaphoreType.DMA((2,)),
        ],
    )
    return f(p1, p2, src3d, dst3d)


BLK_E = 2000
E_GRID = N_EDGES // BLK_E


def _edge_term(zpart, ef, w3, bvec):
    return zpart + jnp.dot(ef, w3, preferred_element_type=jnp.float32) + bvec


def _stats_body(zpart_ref, ef_ref, w3_ref, b_ref, out_ref, acc):
    i = pl.program_id(0)

    @pl.when(i == 0)
    def _():
        acc[...] = jnp.zeros_like(acc)

    z = _edge_term(zpart_ref[...], ef_ref[...], w3_ref[...], b_ref[...])
    acc[0:1, :] += jnp.sum(z, axis=0, keepdims=True)
    acc[1:2, :] += jnp.sum(z * z, axis=0, keepdims=True)

    @pl.when(i == E_GRID - 1)
    def _():
        out_ref[...] = acc[...]


def _stats(zpart, ef, w3, bvec):
    return pl.pallas_call(
        _stats_body,
        grid=(E_GRID,),
        in_specs=[
            pl.BlockSpec((BLK_E, OUT_DIM), lambda i: (i, 0)),
            pl.BlockSpec((BLK_E, E_DIM), lambda i: (i, 0)),
            pl.BlockSpec((E_DIM, OUT_DIM), lambda i: (0, 0)),
            pl.BlockSpec((1, OUT_DIM), lambda i: (0, 0)),
        ],
        out_specs=pl.BlockSpec((8, OUT_DIM), lambda i: (0, 0)),
        out_shape=jax.ShapeDtypeStruct((8, OUT_DIM), jnp.float32),
        scratch_shapes=[pltpu.VMEM((8, OUT_DIM), jnp.float32)],
    )(zpart, ef, w3, bvec)


def _msg_body(zpart_ref, ef_ref, w3_ref, b_ref, stats_ref, gam_ref, bet_ref,
              msg_ref):
    z = _edge_term(zpart_ref[...], ef_ref[...], w3_ref[...], b_ref[...])
    mean = stats_ref[0:1, :] * (1.0 / N_EDGES)
    var = stats_ref[1:2, :] * (1.0 / N_EDGES) - mean * mean
    scale = gam_ref[...] * lax.rsqrt(var + 1e-5)
    shift = bet_ref[...] - mean * scale
    zn = z * scale + shift
    sig = jax.nn.sigmoid(zn[:, :H])
    xp = zn[:, H:]
    sp = jnp.maximum(xp, 0.0) + jnp.log1p(jnp.exp(-jnp.abs(xp)))
    msg_ref[...] = sig * sp


def _msg(zpart, ef, w3, bvec, stats, gamma, beta):
    return pl.pallas_call(
        _msg_body,
        grid=(E_GRID,),
        in_specs=[
            pl.BlockSpec((BLK_E, OUT_DIM), lambda i: (i, 0)),
            pl.BlockSpec((BLK_E, E_DIM), lambda i: (i, 0)),
            pl.BlockSpec((E_DIM, OUT_DIM), lambda i: (0, 0)),
            pl.BlockSpec((1, OUT_DIM), lambda i: (0, 0)),
            pl.BlockSpec((8, OUT_DIM), lambda i: (0, 0)),
            pl.BlockSpec((1, OUT_DIM), lambda i: (0, 0)),
            pl.BlockSpec((1, OUT_DIM), lambda i: (0, 0)),
        ],
        out_specs=pl.BlockSpec((BLK_E, H), lambda i: (i, 0)),
        out_shape=jax.ShapeDtypeStruct((N_EDGES, H), jnp.float32),
    )(zpart, ef, w3, bvec, stats, gamma, beta)


N_NODES_PAD = 10240
ROWS_PER_TILE = N_NODES_PAD // NS


def _scatter_body(msg_hbm, src_hbm, zeros_hbm, parts_hbm, sidx, mbuf, acc, sem):
    cid = lax.axis_index("c")
    sid = lax.axis_index("s")
    wid = sid * NC + cid
    base = wid * EDGES_PER_TILE
    stripe = sid * ROWS_PER_TILE

    pltpu.sync_copy(src_hbm.at[wid], sidx)
    pltpu.sync_copy(zeros_hbm.at[pl.ds(stripe, ROWS_PER_TILE)],
                    acc.at[pl.ds(stripe, ROWS_PER_TILE)])
    plsc.subcore_barrier()

    def fire(j, s):
        pltpu.async_copy(msg_hbm.at[pl.ds(base + j * CHUNK, CHUNK)],
                         mbuf.at[s], sem.at[s])

    def process(j, s):
        pltpu.make_async_copy(msg_hbm.at[pl.ds(base + j * CHUNK, CHUNK)],
                              mbuf.at[s], sem.at[s]).wait()
        pltpu.sync_copy(mbuf.at[s], acc.at[sidx.at[j]], add=True)

    fire(0, 0)
    fire(1, 1)

    def chunk(j2, carry):
        for b in range(2):
            j = 2 * j2 + b
            process(j, b)

            @pl.when(j < NCHUNK - 2)
            def _():
                fire(j + 2, b)
        return carry

    lax.fori_loop(0, (NCHUNK - 1) // 2, chunk, 0)
    process(NCHUNK - 1, (NCHUNK - 1) % 2)
    plsc.subcore_barrier()
    pltpu.sync_copy(acc.at[pl.ds(stripe, ROWS_PER_TILE)],
                    parts_hbm.at[cid, pl.ds(stripe, ROWS_PER_TILE)])


def _sc_scatter(msg, src3d, zeros):
    mesh = plsc.VectorSubcoreMesh(core_axis_name="c", subcore_axis_name="s")
    f = pl.kernel(
        _scatter_body,
        out_type=jax.ShapeDtypeStruct((NC, N_NODES_PAD, H), jnp.float32),
        mesh=mesh,
        scratch_types=[
            pltpu.VMEM((NCHUNK, CHUNK), jnp.int32),
            pltpu.VMEM((2, CHUNK, H), jnp.float32),
            pltpu.VMEM_SHARED((N_NODES_PAD, H), jnp.float32),
            pltpu.SemaphoreType.DMA((2,)),
        ],
    )
    return f(msg, src3d, zeros)


def _final_body(nf_ref, p0_ref, p1_ref, out_ref):
    out_ref[...] = nf_ref[...] + p0_ref[0] + p1_ref[0]


def _final_add(nf, parts):
    grid = N_NODES // BLK_N
    return pl.pallas_call(
        _final_body,
        grid=(grid,),
        in_specs=[
            pl.BlockSpec((BLK_N, H), lambda i: (i, 0)),
            pl.BlockSpec((1, BLK_N, H), lambda i: (0, i, 0)),
            pl.BlockSpec((1, BLK_N, H), lambda i: (1, i, 0)),
        ],
        out_specs=pl.BlockSpec((BLK_N, H), lambda i: (i, 0)),
        out_shape=jax.ShapeDtypeStruct((N_NODES, H), jnp.float32),
    )(nf, parts, parts)


@jax.jit
def kernel(node_feats, edge_index, edge_feats, W, b, gamma, beta):
    src = edge_index[0].astype(jnp.int32)
    dst = edge_index[1].astype(jnp.int32)
    wt = W.T
    w12 = jnp.concatenate([wt[:H], wt[H:2 * H]], axis=1)
    w3 = wt[2 * H:]
    bvec = b.reshape(1, OUT_DIM)
    gam = gamma.reshape(1, OUT_DIM)
    bet = beta.reshape(1, OUT_DIM)
    src3d = src.reshape(NW, NCHUNK, CHUNK)
    dst3d = dst.reshape(NW, NCHUNK, CHUNK)

    p1, p2 = _node_matmul(node_feats, w12)
    zpart = _sc_gather(p1, p2, src3d, dst3d)
    stats = _stats(zpart, edge_feats, w3, bvec)
    msg = _msg(zpart, edge_feats, w3, bvec, stats, gam, bet)
    zeros = jnp.zeros((N_NODES_PAD, H), jnp.float32)
    parts = _sc_scatter(msg, src3d, zeros)
    return _final_add(node_feats, parts)
